# baseline (speedup 1.0000x reference)
import functools

import jax
import jax.numpy as jnp
from jax import lax
from jax.experimental import pallas as pl
from jax.experimental.pallas import tpu as pltpu
from jax.experimental.pallas import tpu_sc as plsc

N_ATOMS = 100000
N_TYPES = 100
N_PROPS = 128
N_SYSTEMS = 1024

_NC = 2
_NS = 16
_NW = _NC * _NS

_S = 8
_A = _NW // _S
_R = N_SYSTEMS // _S
_TPAD = 128

_CHUNK = 25088
_LAST = N_ATOMS - (_A - 1) * _CHUNK
_FB = 256
_NB = _CHUNK // _FB
_CP = 8192
_MAXCP = 4
_STAGE = _MAXCP * _CP
_NPAD = (_A - 1) * _CHUNK + _NB * _FB + _CP
_FPAD = _NPAD // _FB
_IMAX = 2147483647


def _sc_histogram(packed, fence):
    mesh = plsc.VectorSubcoreMesh(core_axis_name="c", subcore_axis_name="s")

    @functools.partial(
        pl.kernel,
        mesh=mesh,
        out_type=jax.ShapeDtypeStruct((_NW, _R, _TPAD), jnp.float32),
        scratch_types=[
            pltpu.VMEM((_STAGE + 16,), jnp.int32),
            pltpu.VMEM((112 + 16,), jnp.int32),
            pltpu.VMEM((_R, _TPAD), jnp.float32),
        ],
        compiler_params=pltpu.CompilerParams(needs_layout_passes=False),
    )
    def hist_kernel(packed_hbm, fence_hbm, out_hbm, pk_v, fence_v, hist_v):
        wid = lax.axis_index("s") * _NC + lax.axis_index("c")
        shard = wid % _A
        lo = (wid // _A) * _R
        base = shard * _CHUNK
        n = jnp.where(shard == _A - 1, _LAST, _CHUNK)
        nb = (n + _FB - 1) // _FB

        pltpu.sync_copy(fence_hbm.at[pl.ds(shard * 112, 112)],
                        fence_v.at[pl.ds(0, 112)])

        def fence_bound(bound):
            def body(_, ab):
                a, b = ab
                m = (a + b) // 2
                go_right = fence_v[pl.ds(m, 16)][0] < bound
                return (jnp.where(go_right, m + 1, a),
                        jnp.where(go_right, b, m))
            a, _b = lax.fori_loop(0, 8, body, (0, nb))
            return jnp.minimum(a, nb)

        c_lo = fence_bound(lo * 128)
        c_hi = fence_bound((lo + _R) * 128)
        b0 = jnp.maximum(c_lo - 1, 0)
        ncp = jnp.maximum((c_hi - b0 + 31) // 32, 1)

        def copy_body(j, carry):
            pltpu.sync_copy(
                packed_hbm.at[pl.ds(base + b0 * _FB + j * _CP, _CP)],
                pk_v.at[pl.ds(j * _CP, _CP)])
            return carry

        lax.fori_loop(0, ncp, copy_body, 0)

        zeros = jnp.zeros((16,), jnp.float32)
        ones = jnp.ones((16,), jnp.float32)

        def zero_body(i, carry):
            for j in range(_TPAD // 16):
                hist_v[i, pl.ds(j * 16, 16)] = zeros
            return carry

        lax.fori_loop(0, _R, zero_body, 0)

        sv = n - b0 * _FB
        cap = jnp.minimum(ncp * _CP, sv)

        def stage_bound(bound):
            def body(_, ab):
                a, b = ab
                m = (a + b) // 2
                go_right = pk_v[pl.ds(m, 16)][0] < bound
                return (jnp.where(go_right, m + 1, a),
                        jnp.where(go_right, b, m))
            a, _b = lax.fori_loop(0, 16, body, (0, cap))
            return jnp.minimum(a, cap)

        i_lo = stage_bound(lo * 128)
        i_hi = stage_bound((lo + _R) * 128)

        lane = lax.broadcasted_iota(jnp.int32, (16,), 0)

        def atom_body(i, carry):
            o = i * 16
            p = pk_v[pl.ds(o, 16)]
            t = jnp.bitwise_and(p, 127)
            s = jnp.right_shift(p, 7) - lo
            mask = jnp.logical_and(
                jnp.logical_and(s >= 0, s < _R), o + lane < sv)
            plsc.addupdate_scatter(hist_v, [s, t], ones, mask=mask)
            return carry

        b_lo = jnp.maximum(i_lo // 16 - 1, 0)
        b_hi = jnp.minimum(i_hi // 16 + 1, cap // 16)
        lax.fori_loop(b_lo, b_hi, atom_body, 0)

        pltpu.sync_copy(hist_v, out_hbm.at[wid])

    return hist_kernel(packed, fence)


def _tc_body(counts_ref, tti_ref, w_ref, o_ref):
    acc = jnp.sum(counts_ref[...], axis=1).reshape(N_SYSTEMS, _TPAD)
    tti = tti_ref[...]
    onehot = (tti == lax.broadcasted_iota(jnp.int32, (_TPAD, _TPAD), 1)
              ).astype(jnp.float32)
    w_eff = jnp.dot(onehot, w_ref[...], preferred_element_type=jnp.float32,
                    precision=lax.Precision.HIGHEST)
    o_ref[...] = jnp.dot(acc, w_eff, preferred_element_type=jnp.float32,
                         precision=lax.Precision.HIGHEST)


def _tc_reduce_matmul(counts, tti_pad, w_pad):
    return pl.pallas_call(
        _tc_body,
        out_shape=jax.ShapeDtypeStruct((N_SYSTEMS, N_PROPS), jnp.float32),
    )(counts.reshape(_S, _A, _R, _TPAD), tti_pad, w_pad)


def kernel(atom_types, system_ids, type_to_index, weights):
    packed = (system_ids.astype(jnp.int32) * 128
              + atom_types.astype(jnp.int32))
    packed_pad = jnp.full((_NPAD,), _IMAX, jnp.int32).at[:N_ATOMS].set(packed)
    fence = packed_pad[::_FB]
    fence_rows = jnp.concatenate(
        [fence[_NB * s:_NB * s + 112] for s in range(_A)])
    counts = _sc_histogram(packed_pad, fence_rows)
    tti_pad = jnp.concatenate(
        [type_to_index.astype(jnp.int32),
         jnp.full((_TPAD - N_TYPES,), -1, jnp.int32)]).reshape(_TPAD, 1)
    w_pad = jnp.zeros((_TPAD, N_PROPS), jnp.float32).at[:N_TYPES].set(
        weights.astype(jnp.float32))
    return _tc_reduce_matmul(counts, tti_pad, w_pad)


# speedup vs baseline: 1.0370x; 1.0370x over previous
import functools

import jax
import jax.numpy as jnp
from jax import lax
from jax.experimental import pallas as pl
from jax.experimental.pallas import tpu as pltpu
from jax.experimental.pallas import tpu_sc as plsc

N_ATOMS = 100000
N_TYPES = 100
N_PROPS = 128
N_SYSTEMS = 1024

_NC = 2
_NS = 16
_NW = _NC * _NS

_S = 8
_A = _NW // _S
_R = N_SYSTEMS // _S
_TPAD = 128

_CHUNK = 25024
_LAST = N_ATOMS - (_A - 1) * _CHUNK


def _sc_histogram(packed):
    mesh = plsc.VectorSubcoreMesh(core_axis_name="c", subcore_axis_name="s")

    @functools.partial(
        pl.kernel,
        mesh=mesh,
        out_type=jax.ShapeDtypeStruct((_NW, _R, _TPAD), jnp.float32),
        scratch_types=[
            pltpu.VMEM((_CHUNK + 16,), jnp.int32),
            pltpu.VMEM((_R, _TPAD), jnp.float32),
        ],
        compiler_params=pltpu.CompilerParams(needs_layout_passes=False),
    )
    def hist_kernel(packed_hbm, out_hbm, pk_v, hist_v):
        wid = lax.axis_index("s") * _NC + lax.axis_index("c")
        shard = wid % _A
        lo = (wid // _A) * _R
        base = shard * _CHUNK
        is_last = shard == _A - 1
        n = jnp.where(is_last, _LAST, _CHUNK)

        @pl.when(jnp.logical_not(is_last))
        def _():
            pltpu.sync_copy(packed_hbm.at[pl.ds(base, _CHUNK)],
                            pk_v.at[pl.ds(0, _CHUNK)])

        @pl.when(is_last)
        def _():
            pltpu.sync_copy(packed_hbm.at[pl.ds(base, _LAST)],
                            pk_v.at[pl.ds(0, _LAST)])

        zeros = jnp.zeros((16,), jnp.float32)
        ones = jnp.ones((16,), jnp.float32)

        def zero_body(i, carry):
            for j in range(_TPAD // 16):
                hist_v[i, pl.ds(j * 16, 16)] = zeros
            return carry

        lax.fori_loop(0, _R, zero_body, 0)

        def lower_bound(bound):
            def body(_, ab):
                a, b = ab
                m = (a + b) // 2
                go_right = pk_v[pl.ds(m, 16)][0] < bound
                return (jnp.where(go_right, m + 1, a),
                        jnp.where(go_right, b, m))
            a, _b = lax.fori_loop(0, 15, body, (0, n))
            return a

        i_lo = jnp.minimum(lower_bound(lo * 128), n)
        i_hi = jnp.minimum(lower_bound((lo + _R) * 128), n)

        def atom_body(i, carry):
            p = pk_v[pl.ds(i * 16, 16)]
            t = jnp.bitwise_and(p, 127)
            s = jnp.right_shift(p, 7) - lo
            mask = jnp.logical_and(s >= 0, s < _R)
            plsc.addupdate_scatter(hist_v, [s, t], ones, mask=mask)
            return carry

        b_lo = jnp.maximum(i_lo // 16 - 1, 0)
        b_hi = jnp.minimum(i_hi // 16 + 1, n // 16)
        lax.fori_loop(b_lo, b_hi, atom_body, 0)

        pltpu.sync_copy(hist_v, out_hbm.at[wid])

    return hist_kernel(packed)


def _tc_body(counts_ref, tti_ref, w_ref, o_ref):
    acc = jnp.sum(counts_ref[...], axis=1).reshape(N_SYSTEMS, _TPAD)
    tti = tti_ref[...]
    onehot = (tti == lax.broadcasted_iota(jnp.int32, (_TPAD, _TPAD), 1)
              ).astype(jnp.float32)
    w_eff = jnp.dot(onehot, w_ref[...], preferred_element_type=jnp.float32,
                    precision=lax.Precision.HIGHEST)
    o_ref[...] = jnp.dot(acc, w_eff, preferred_element_type=jnp.float32,
                         precision=lax.Precision.HIGHEST)


def _tc_reduce_matmul(counts, tti_pad, w_pad):
    return pl.pallas_call(
        _tc_body,
        out_shape=jax.ShapeDtypeStruct((N_SYSTEMS, N_PROPS), jnp.float32),
    )(counts.reshape(_S, _A, _R, _TPAD), tti_pad, w_pad)


def kernel(atom_types, system_ids, type_to_index, weights):
    packed = (system_ids.astype(jnp.int32) * 128
              + atom_types.astype(jnp.int32))
    counts = _sc_histogram(packed)
    tti_pad = jnp.concatenate(
        [type_to_index.astype(jnp.int32),
         jnp.full((_TPAD - N_TYPES,), -1, jnp.int32)]).reshape(_TPAD, 1)
    w_pad = jnp.zeros((_TPAD, N_PROPS), jnp.float32).at[:N_TYPES].set(
        weights.astype(jnp.float32))
    return _tc_reduce_matmul(counts, tti_pad, w_pad)
